# baseline (speedup 1.0000x reference)
import functools

import jax
import jax.numpy as jnp
from jax import lax
from jax.experimental import pallas as pl
from jax.experimental.pallas import tpu as pltpu
from jax.experimental.pallas import tpu_sc as plsc

NC = 2
NS = 16
NW = NC * NS
LANES = 16
EC = 125
CPS = 8


def _sc_mesh():
    return plsc.VectorSubcoreMesh(core_axis_name="c", subcore_axis_name="s")


def _degrees(ei4, n_pad):
    _, _, nchunk, _ = ei4.shape
    rpt = n_pad // NS
    fs = 5
    assert nchunk % fs == 0

    @functools.partial(
        pl.kernel,
        out_type=jax.ShapeDtypeStruct((NC, 2, n_pad), jnp.float32),
        mesh=_sc_mesh(),
        scratch_types=[
            pltpu.VMEM((nchunk, EC), jnp.int32),
            pltpu.VMEM((nchunk, EC), jnp.int32),
            pltpu.VMEM((EC,), jnp.float32),
            pltpu.VMEM((rpt,), jnp.float32),
            pltpu.VMEM_SHARED((n_pad,), jnp.float32),
            pltpu.VMEM_SHARED((n_pad,), jnp.float32),
            pltpu.SemaphoreType.DMA,
        ],
    )
    def deg_k(ei_hbm, out_hbm, sidx_v, didx_v, ones_v, stage_v,
              dout_sh, din_sh, sem):
        cid = lax.axis_index("c")
        sid = lax.axis_index("s")
        wid = cid * NS + sid
        pltpu.sync_copy(ei_hbm.at[0, wid], sidx_v)
        pltpu.sync_copy(ei_hbm.at[1, wid], didx_v)
        for i in range(EC // LANES):
            ones_v[pl.ds(i * LANES, LANES)] = jnp.ones((LANES,), jnp.float32)
        ones_v[pl.ds(EC - LANES, LANES)] = jnp.ones((LANES,), jnp.float32)
        for i in range(rpt // LANES):
            stage_v[pl.ds(i * LANES, LANES)] = jnp.zeros((LANES,), jnp.float32)
        r0 = sid * rpt
        pltpu.sync_copy(stage_v, dout_sh.at[pl.ds(r0, rpt)])
        pltpu.sync_copy(stage_v, din_sh.at[pl.ds(r0, rpt)])
        plsc.subcore_barrier()

        def flight(f, _):
            cps = []
            for j in range(fs):
                k = f * fs + j
                cps.append(pltpu.async_copy(ones_v, dout_sh.at[sidx_v.at[k]], sem, add=True))
                cps.append(pltpu.async_copy(ones_v, din_sh.at[didx_v.at[k]], sem, add=True))
            for cp in cps:
                cp.wait()
            return ()

        lax.fori_loop(0, nchunk // fs, flight, ())
        plsc.subcore_barrier()
        pltpu.sync_copy(dout_sh.at[pl.ds(r0, rpt)], stage_v)
        pltpu.sync_copy(stage_v, out_hbm.at[cid, 0, pl.ds(r0, rpt)])
        pltpu.sync_copy(din_sh.at[pl.ds(r0, rpt)], stage_v)
        pltpu.sync_copy(stage_v, out_hbm.at[cid, 1, pl.ds(r0, rpt)])

    return deg_k(ei4)


def _aggregate(h, ei4, n_pad):
    n, d = h.shape
    _, _, nchunk, _ = ei4.shape
    nsb = nchunk // CPS
    rpt = n_pad // NS
    assert nchunk == nsb * CPS and nsb % 2 == 0 and CPS % 2 == 0

    @functools.partial(
        pl.kernel,
        out_type=jax.ShapeDtypeStruct((NC, n_pad, d), jnp.float32),
        mesh=_sc_mesh(),
        scratch_types=[
            pltpu.VMEM((2, CPS, EC), jnp.int32),
            pltpu.VMEM((2, CPS, EC), jnp.int32),
            pltpu.VMEM((2, EC, d), jnp.float32),
            pltpu.VMEM_SHARED((n_pad, d), jnp.float32),
            pltpu.SemaphoreType.DMA,
            pltpu.SemaphoreType.DMA,
            pltpu.SemaphoreType.DMA,
            pltpu.SemaphoreType.DMA,
            pltpu.SemaphoreType.DMA,
            pltpu.SemaphoreType.DMA,
        ],
    )
    def agg_k(h_hbm, ei_hbm, out_hbm, sibuf, dibuf, rows_v, acc_sh,
              isem0, isem1, gsem0, gsem1, ssem0, ssem1):
        isem = (isem0, isem1)
        gsem = (gsem0, gsem1)
        ssem = (ssem0, ssem1)
        cid = lax.axis_index("c")
        sid = lax.axis_index("s")
        wid = cid * NS + sid

        def zrow(r, _):
            for j in range(d // LANES):
                rows_v[0, r, pl.ds(j * LANES, LANES)] = jnp.zeros((LANES,), jnp.float32)
            return ()

        lax.fori_loop(0, EC, zrow, ())
        r0 = sid * rpt
        zc = EC
        off = 0
        while off < rpt:
            sz = min(zc, rpt - off)
            pltpu.sync_copy(rows_v.at[0, pl.ds(0, sz)],
                            acc_sh.at[pl.ds(r0 + off, sz)])
            off += sz
        plsc.subcore_barrier()

        def load_sb(sb, t):
            pltpu.async_copy(ei_hbm.at[0, wid, pl.ds(sb * CPS, CPS)], sibuf.at[t], isem[t])
            pltpu.async_copy(ei_hbm.at[1, wid, pl.ds(sb * CPS, CPS)], dibuf.at[t], isem[t])

        def wait_sb(t):
            pltpu.make_async_copy(ei_hbm.at[0, 0, pl.ds(0, CPS)], sibuf.at[t], isem[t]).wait()
            pltpu.make_async_copy(ei_hbm.at[0, 0, pl.ds(0, CPS)], dibuf.at[t], isem[t]).wait()

        def issue_gather(t, j, s):
            pltpu.async_copy(h_hbm.at[sibuf.at[t, j]], rows_v.at[s], gsem[s])

        def wait_gather(t, j, s):
            pltpu.make_async_copy(h_hbm.at[sibuf.at[t, j]], rows_v.at[s], gsem[s]).wait()

        def issue_scatter(t, j, s):
            pltpu.async_copy(rows_v.at[s], acc_sh.at[dibuf.at[t, j]], ssem[s], add=True)

        def wait_scatter(t, j, s):
            pltpu.make_async_copy(rows_v.at[s], acc_sh.at[dibuf.at[t, j]], ssem[s]).wait()

        def run_sb(sb, t, first_sb, last_sb):
            for j in range(CPS):
                s = j % 2
                first_k = first_sb and j == 0
                last_k = last_sb and j == CPS - 1
                if not first_k:
                    if j == 0:
                        wait_scatter(1 - t, CPS - 1, 1 - s)
                    else:
                        wait_scatter(t, j - 1, 1 - s)
                wait_gather(t, j, s)
                issue_scatter(t, j, s)
                if not last_k:
                    if j == CPS - 1:
                        wait_sb(1 - t)
                        issue_gather(1 - t, 0, 1 - s)
                    else:
                        issue_gather(t, j + 1, 1 - s)
                if j == 0 and not last_sb:
                    load_sb(sb + 1, 1 - t)

        pltpu.sync_copy(ei_hbm.at[0, wid, pl.ds(0, CPS)], sibuf.at[0])
        pltpu.sync_copy(ei_hbm.at[1, wid, pl.ds(0, CPS)], dibuf.at[0])
        issue_gather(0, 0, 0)
        run_sb(0, 0, True, False)

        def pair(q, _):
            run_sb(2 * q + 1, 1, False, False)
            run_sb(2 * q + 2, 0, False, False)
            return ()

        lax.fori_loop(0, (nsb - 2) // 2, pair, ())
        run_sb(nsb - 1, 1, False, True)
        wait_scatter(1, CPS - 1, (CPS - 1) % 2)
        plsc.subcore_barrier()
        pltpu.sync_copy(acc_sh.at[pl.ds(r0, rpt)], out_hbm.at[cid, pl.ds(r0, rpt)])

    return agg_k(h, ei4)


def _norms(degp, br=512):
    n_pad = degp.shape[2]
    eye = jnp.eye(br, dtype=jnp.float32)

    def body(deg_ref, eye_ref, ns_ref, nd_ref):
        dg = deg_ref[...]
        deg = jnp.clip(dg[0] + dg[1], 1.0, None)
        norm = lax.rsqrt(deg)
        cols = lax.dot_general(eye_ref[...], norm,
                               (((1,), (1,)), ((), ())),
                               precision=lax.Precision.HIGHEST,
                               preferred_element_type=jnp.float32)
        ns_ref[...] = cols[:, 0:1]
        nd_ref[...] = cols[:, 1:2]

    return pl.pallas_call(
        body,
        grid=(n_pad // br,),
        in_specs=[
            pl.BlockSpec((2, 2, br), lambda i: (0, 0, i)),
            pl.BlockSpec((br, br), lambda i: (0, 0)),
        ],
        out_specs=[
            pl.BlockSpec((br, 1), lambda i: (i, 0)),
            pl.BlockSpec((br, 1), lambda i: (i, 0)),
        ],
        out_shape=[
            jax.ShapeDtypeStruct((n_pad, 1), jnp.float32),
            jax.ShapeDtypeStruct((n_pad, 1), jnp.float32),
        ],
    )(degp, eye)


def _scale(x, ns_col, br=2000):
    n, d = x.shape

    def body(x_ref, ns_ref, h0_ref):
        h0_ref[...] = x_ref[...] * ns_ref[...]

    return pl.pallas_call(
        body,
        grid=(n // br,),
        in_specs=[
            pl.BlockSpec((br, d), lambda i: (i, 0)),
            pl.BlockSpec((br, 1), lambda i: (i, 0)),
        ],
        out_specs=pl.BlockSpec((br, d), lambda i: (i, 0)),
        out_shape=jax.ShapeDtypeStruct((n, d), jnp.float32),
    )(x, ns_col)


def _layer(aggp, nd_col, w, b2d, ns_col, relu, out_dtype, n, br=1000):
    d = w.shape[0]
    scaled = ns_col is not None

    def body(agg_ref, nd_ref, w_ref, b_ref, *rest):
        if scaled:
            ns_ref, o_ref = rest
        else:
            (o_ref,) = rest
        agg = (agg_ref[0] + agg_ref[1]) * nd_ref[...]
        h = jnp.dot(agg, w_ref[...], preferred_element_type=jnp.float32) + b_ref[...]
        if relu:
            h = jnp.maximum(h, 0.0)
        if scaled:
            h = h * ns_ref[...]
        o_ref[...] = h.astype(o_ref.dtype)

    in_specs = [
        pl.BlockSpec((NC, br, d), lambda i: (0, i, 0)),
        pl.BlockSpec((br, 1), lambda i: (i, 0)),
        pl.BlockSpec((d, d), lambda i: (0, 0)),
        pl.BlockSpec((1, d), lambda i: (0, 0)),
    ]
    args = [aggp, nd_col, w, b2d]
    if scaled:
        in_specs.append(pl.BlockSpec((br, 1), lambda i: (i, 0)))
        args.append(ns_col)
    return pl.pallas_call(
        body,
        grid=(n // br,),
        in_specs=in_specs,
        out_specs=pl.BlockSpec((br, d), lambda i: (i, 0)),
        out_shape=jax.ShapeDtypeStruct((n, d), out_dtype),
    )(*args)


def kernel(inputs, edge_index, W1, b1, W2, b2):
    x = inputs
    n, d = x.shape
    e = edge_index.shape[1]
    ei4 = edge_index.reshape(2, NW, e // (NW * EC), EC)
    n_pad = -(-n // (NS * LANES)) * (NS * LANES)

    degp = _degrees(ei4, n_pad)
    ns, nd = _norms(degp)
    h0 = _scale(x, ns)
    p1 = _aggregate(h0, ei4, n_pad)
    h1 = _layer(p1, nd, W1, b1.reshape(1, d), ns, relu=True,
                out_dtype=jnp.float32, n=n)
    p2 = _aggregate(h1, ei4, n_pad)
    out = _layer(p2, nd, W2, b2.reshape(1, d), None, relu=False,
                 out_dtype=jnp.float32, n=n)
    return out


# speedup vs baseline: 1.1420x; 1.1420x over previous
import functools

import jax
import jax.numpy as jnp
from jax import lax
from jax.experimental import pallas as pl
from jax.experimental.pallas import tpu as pltpu
from jax.experimental.pallas import tpu_sc as plsc

NC = 2
NS = 16
NW = NC * NS
LANES = 16
EC = 50
CPS = 8


def _sc_mesh():
    return plsc.VectorSubcoreMesh(core_axis_name="c", subcore_axis_name="s")


def _degrees(ei4, n_pad):
    _, _, nchunk, _ = ei4.shape
    rpt = n_pad // NS
    fs = 5
    assert nchunk % fs == 0

    @functools.partial(
        pl.kernel,
        out_type=jax.ShapeDtypeStruct((NC, 2, n_pad), jnp.float32),
        mesh=_sc_mesh(),
        scratch_types=[
            pltpu.VMEM((nchunk, EC), jnp.int32),
            pltpu.VMEM((nchunk, EC), jnp.int32),
            pltpu.VMEM((EC,), jnp.float32),
            pltpu.VMEM((rpt,), jnp.float32),
            pltpu.VMEM_SHARED((n_pad,), jnp.float32),
            pltpu.VMEM_SHARED((n_pad,), jnp.float32),
            pltpu.SemaphoreType.DMA,
        ],
    )
    def deg_k(ei_hbm, out_hbm, sidx_v, didx_v, ones_v, stage_v,
              dout_sh, din_sh, sem):
        cid = lax.axis_index("c")
        sid = lax.axis_index("s")
        wid = cid * NS + sid
        pltpu.sync_copy(ei_hbm.at[0, wid], sidx_v)
        pltpu.sync_copy(ei_hbm.at[1, wid], didx_v)
        for i in range(EC // LANES):
            ones_v[pl.ds(i * LANES, LANES)] = jnp.ones((LANES,), jnp.float32)
        ones_v[pl.ds(EC - LANES, LANES)] = jnp.ones((LANES,), jnp.float32)
        for i in range(rpt // LANES):
            stage_v[pl.ds(i * LANES, LANES)] = jnp.zeros((LANES,), jnp.float32)
        r0 = sid * rpt
        pltpu.sync_copy(stage_v, dout_sh.at[pl.ds(r0, rpt)])
        pltpu.sync_copy(stage_v, din_sh.at[pl.ds(r0, rpt)])
        plsc.subcore_barrier()

        def flight(f, _):
            cps = []
            for j in range(fs):
                k = f * fs + j
                cps.append(pltpu.async_copy(ones_v, dout_sh.at[sidx_v.at[k]], sem, add=True))
                cps.append(pltpu.async_copy(ones_v, din_sh.at[didx_v.at[k]], sem, add=True))
            for cp in cps:
                cp.wait()
            return ()

        lax.fori_loop(0, nchunk // fs, flight, ())
        plsc.subcore_barrier()
        pltpu.sync_copy(dout_sh.at[pl.ds(r0, rpt)], stage_v)
        pltpu.sync_copy(stage_v, out_hbm.at[cid, 0, pl.ds(r0, rpt)])
        pltpu.sync_copy(din_sh.at[pl.ds(r0, rpt)], stage_v)
        pltpu.sync_copy(stage_v, out_hbm.at[cid, 1, pl.ds(r0, rpt)])

    return deg_k(ei4)


def _aggregate(h, ei4, n_pad):
    n, d = h.shape
    _, _, nchunk, _ = ei4.shape
    nsb = nchunk // CPS
    rpt = n_pad // NS
    assert nchunk == nsb * CPS and CPS % 4 == 0 and nsb >= 3 and (nsb - 3) % 2 == 0

    @functools.partial(
        pl.kernel,
        out_type=jax.ShapeDtypeStruct((NC, n_pad, d), jnp.float32),
        mesh=_sc_mesh(),
        scratch_types=[
            pltpu.VMEM((2, CPS, EC), jnp.int32),
            pltpu.VMEM((2, CPS, EC), jnp.int32),
            pltpu.VMEM((4, EC, d), jnp.float32),
            pltpu.VMEM_SHARED((n_pad, d), jnp.float32),
            pltpu.SemaphoreType.DMA,
            pltpu.SemaphoreType.DMA,
            pltpu.SemaphoreType.DMA,
            pltpu.SemaphoreType.DMA,
            pltpu.SemaphoreType.DMA,
            pltpu.SemaphoreType.DMA,
            pltpu.SemaphoreType.DMA,
            pltpu.SemaphoreType.DMA,
            pltpu.SemaphoreType.DMA,
            pltpu.SemaphoreType.DMA,
        ],
    )
    def agg_k(h_hbm, ei_hbm, out_hbm, sibuf, dibuf, rows_v, acc_sh,
              isem0, isem1, gsem0, gsem1, gsem2, gsem3, ssem0, ssem1, ssem2, ssem3):
        isem = (isem0, isem1)
        gsem = (gsem0, gsem1, gsem2, gsem3)
        ssem = (ssem0, ssem1, ssem2, ssem3)
        cid = lax.axis_index("c")
        sid = lax.axis_index("s")
        wid = cid * NS + sid

        def zrow(r, _):
            for j in range(d // LANES):
                rows_v[0, r, pl.ds(j * LANES, LANES)] = jnp.zeros((LANES,), jnp.float32)
            return ()

        lax.fori_loop(0, EC, zrow, ())
        r0 = sid * rpt
        off = 0
        while off < rpt:
            sz = min(EC, rpt - off)
            pltpu.sync_copy(rows_v.at[0, pl.ds(0, sz)],
                            acc_sh.at[pl.ds(r0 + off, sz)])
            off += sz
        plsc.subcore_barrier()

        def load_sb(sb, t):
            pltpu.async_copy(ei_hbm.at[0, wid, pl.ds(sb * CPS, CPS)], sibuf.at[t], isem[t])
            pltpu.async_copy(ei_hbm.at[1, wid, pl.ds(sb * CPS, CPS)], dibuf.at[t], isem[t])

        def wait_sb(t):
            pltpu.make_async_copy(ei_hbm.at[0, 0, pl.ds(0, CPS)], sibuf.at[t], isem[t]).wait()
            pltpu.make_async_copy(ei_hbm.at[0, 0, pl.ds(0, CPS)], dibuf.at[t], isem[t]).wait()

        def issue_gather(t, j, s):
            pltpu.async_copy(h_hbm.at[sibuf.at[t, j]], rows_v.at[s], gsem[s])

        def wait_gather(t, j, s):
            pltpu.make_async_copy(h_hbm.at[sibuf.at[t, j]], rows_v.at[s], gsem[s]).wait()

        def issue_scatter(t, j, s):
            pltpu.async_copy(rows_v.at[s], acc_sh.at[dibuf.at[t, j]], ssem[s], add=True)

        def wait_scatter(t, j, s):
            pltpu.make_async_copy(rows_v.at[s], acc_sh.at[dibuf.at[t, j]], ssem[s]).wait()

        def run_sb(sb, t, first_sb, last_sb):
            for j in range(CPS):
                s = j % 4
                if not (first_sb and j < 2):
                    if j >= 2:
                        wait_scatter(t, j - 2, (j - 2) % 4)
                    else:
                        wait_scatter(1 - t, j + CPS - 2, (j + CPS - 2) % 4)
                if not (last_sb and j >= CPS - 2):
                    if j == CPS - 2:
                        wait_sb(1 - t)
                        issue_gather(1 - t, 0, (j + 2) % 4)
                    elif j == CPS - 1:
                        issue_gather(1 - t, 1, (j + 2) % 4)
                    else:
                        issue_gather(t, j + 2, (j + 2) % 4)
                wait_gather(t, j, s)
                issue_scatter(t, j, s)
                if j == 1 and not last_sb:
                    load_sb(sb + 1, 1 - t)

        pltpu.sync_copy(ei_hbm.at[0, wid, pl.ds(0, CPS)], sibuf.at[0])
        pltpu.sync_copy(ei_hbm.at[1, wid, pl.ds(0, CPS)], dibuf.at[0])
        issue_gather(0, 0, 0)
        issue_gather(0, 1, 1)
        run_sb(0, 0, True, False)

        def pair(q, _):
            run_sb(2 * q + 1, 1, False, False)
            run_sb(2 * q + 2, 0, False, False)
            return ()

        lax.fori_loop(0, (nsb - 3) // 2, pair, ())
        run_sb(nsb - 2, 1, False, False)
        run_sb(nsb - 1, 0, False, True)
        wait_scatter(0, CPS - 2, (CPS - 2) % 4)
        wait_scatter(0, CPS - 1, (CPS - 1) % 4)
        plsc.subcore_barrier()
        pltpu.sync_copy(acc_sh.at[pl.ds(r0, rpt)], out_hbm.at[cid, pl.ds(r0, rpt)])

    return agg_k(h, ei4)


def _norms(degp, br=512):
    n_pad = degp.shape[2]
    eye = jnp.eye(br, dtype=jnp.float32)

    def body(deg_ref, eye_ref, ns_ref, nd_ref):
        dg = deg_ref[...]
        deg = jnp.clip(dg[0] + dg[1], 1.0, None)
        norm = lax.rsqrt(deg)
        cols = lax.dot_general(eye_ref[...], norm,
                               (((1,), (1,)), ((), ())),
                               precision=lax.Precision.HIGHEST,
                               preferred_element_type=jnp.float32)
        ns_ref[...] = cols[:, 0:1]
        nd_ref[...] = cols[:, 1:2]

    return pl.pallas_call(
        body,
        grid=(n_pad // br,),
        in_specs=[
            pl.BlockSpec((2, 2, br), lambda i: (0, 0, i)),
            pl.BlockSpec((br, br), lambda i: (0, 0)),
        ],
        out_specs=[
            pl.BlockSpec((br, 1), lambda i: (i, 0)),
            pl.BlockSpec((br, 1), lambda i: (i, 0)),
        ],
        out_shape=[
            jax.ShapeDtypeStruct((n_pad, 1), jnp.float32),
            jax.ShapeDtypeStruct((n_pad, 1), jnp.float32),
        ],
    )(degp, eye)


def _scale(x, ns_col, br=2000):
    n, d = x.shape

    def body(x_ref, ns_ref, h0_ref):
        h0_ref[...] = x_ref[...] * ns_ref[...]

    return pl.pallas_call(
        body,
        grid=(n // br,),
        in_specs=[
            pl.BlockSpec((br, d), lambda i: (i, 0)),
            pl.BlockSpec((br, 1), lambda i: (i, 0)),
        ],
        out_specs=pl.BlockSpec((br, d), lambda i: (i, 0)),
        out_shape=jax.ShapeDtypeStruct((n, d), jnp.float32),
    )(x, ns_col)


def _layer(aggp, nd_col, w, b2d, ns_col, relu, out_dtype, n, br=1000):
    d = w.shape[0]
    scaled = ns_col is not None

    def body(agg_ref, nd_ref, w_ref, b_ref, *rest):
        if scaled:
            ns_ref, o_ref = rest
        else:
            (o_ref,) = rest
        agg = (agg_ref[0] + agg_ref[1]) * nd_ref[...]
        h = jnp.dot(agg, w_ref[...], preferred_element_type=jnp.float32) + b_ref[...]
        if relu:
            h = jnp.maximum(h, 0.0)
        if scaled:
            h = h * ns_ref[...]
        o_ref[...] = h.astype(o_ref.dtype)

    in_specs = [
        pl.BlockSpec((NC, br, d), lambda i: (0, i, 0)),
        pl.BlockSpec((br, 1), lambda i: (i, 0)),
        pl.BlockSpec((d, d), lambda i: (0, 0)),
        pl.BlockSpec((1, d), lambda i: (0, 0)),
    ]
    args = [aggp, nd_col, w, b2d]
    if scaled:
        in_specs.append(pl.BlockSpec((br, 1), lambda i: (i, 0)))
        args.append(ns_col)
    return pl.pallas_call(
        body,
        grid=(n // br,),
        in_specs=in_specs,
        out_specs=pl.BlockSpec((br, d), lambda i: (i, 0)),
        out_shape=jax.ShapeDtypeStruct((n, d), out_dtype),
    )(*args)


def kernel(inputs, edge_index, W1, b1, W2, b2):
    x = inputs
    n, d = x.shape
    e = edge_index.shape[1]
    ei4 = edge_index.reshape(2, NW, e // (NW * EC), EC)
    n_pad = -(-n // (NS * LANES)) * (NS * LANES)

    degp = _degrees(ei4, n_pad)
    ns, nd = _norms(degp)
    h0 = _scale(x, ns)
    p1 = _aggregate(h0, ei4, n_pad)
    h1 = _layer(p1, nd, W1, b1.reshape(1, d), ns, relu=True,
                out_dtype=jnp.float32, n=n)
    p2 = _aggregate(h1, ei4, n_pad)
    out = _layer(p2, nd, W2, b2.reshape(1, d), None, relu=False,
                 out_dtype=jnp.float32, n=n)
    return out


# speedup vs baseline: 1.1471x; 1.0045x over previous
import functools

import jax
import jax.numpy as jnp
from jax import lax
from jax.experimental import pallas as pl
from jax.experimental.pallas import tpu as pltpu
from jax.experimental.pallas import tpu_sc as plsc

NC = 2
NS = 16
NW = NC * NS
LANES = 16
EC = 125
CPS = 8


def _sc_mesh():
    return plsc.VectorSubcoreMesh(core_axis_name="c", subcore_axis_name="s")


def _degrees(ei4, n_pad):
    _, _, nchunk, _ = ei4.shape
    rpt = n_pad // NS
    fs = 5
    assert nchunk % fs == 0

    @functools.partial(
        pl.kernel,
        out_type=jax.ShapeDtypeStruct((NC, 2, n_pad), jnp.float32),
        mesh=_sc_mesh(),
        scratch_types=[
            pltpu.VMEM((nchunk, EC), jnp.int32),
            pltpu.VMEM((nchunk, EC), jnp.int32),
            pltpu.VMEM((EC,), jnp.float32),
            pltpu.VMEM((rpt,), jnp.float32),
            pltpu.VMEM_SHARED((n_pad,), jnp.float32),
            pltpu.VMEM_SHARED((n_pad,), jnp.float32),
            pltpu.SemaphoreType.DMA,
        ],
    )
    def deg_k(ei_hbm, out_hbm, sidx_v, didx_v, ones_v, stage_v,
              dout_sh, din_sh, sem):
        cid = lax.axis_index("c")
        sid = lax.axis_index("s")
        wid = cid * NS + sid
        pltpu.sync_copy(ei_hbm.at[0, wid], sidx_v)
        pltpu.sync_copy(ei_hbm.at[1, wid], didx_v)
        for i in range(EC // LANES):
            ones_v[pl.ds(i * LANES, LANES)] = jnp.ones((LANES,), jnp.float32)
        ones_v[pl.ds(EC - LANES, LANES)] = jnp.ones((LANES,), jnp.float32)
        for i in range(rpt // LANES):
            stage_v[pl.ds(i * LANES, LANES)] = jnp.zeros((LANES,), jnp.float32)
        r0 = sid * rpt
        pltpu.sync_copy(stage_v, dout_sh.at[pl.ds(r0, rpt)])
        pltpu.sync_copy(stage_v, din_sh.at[pl.ds(r0, rpt)])
        plsc.subcore_barrier()

        def flight(f, _):
            cps = []
            for j in range(fs):
                k = f * fs + j
                cps.append(pltpu.async_copy(ones_v, dout_sh.at[sidx_v.at[k]], sem, add=True))
                cps.append(pltpu.async_copy(ones_v, din_sh.at[didx_v.at[k]], sem, add=True))
            for cp in cps:
                cp.wait()
            return ()

        lax.fori_loop(0, nchunk // fs, flight, ())
        plsc.subcore_barrier()
        pltpu.sync_copy(dout_sh.at[pl.ds(r0, rpt)], stage_v)
        pltpu.sync_copy(stage_v, out_hbm.at[cid, 0, pl.ds(r0, rpt)])
        pltpu.sync_copy(din_sh.at[pl.ds(r0, rpt)], stage_v)
        pltpu.sync_copy(stage_v, out_hbm.at[cid, 1, pl.ds(r0, rpt)])

    return deg_k(ei4)


def _aggregate(h, ei4, n_pad):
    n, d = h.shape
    _, _, nchunk, _ = ei4.shape
    nsb = nchunk // CPS
    rpt = n_pad // NS
    assert nchunk == nsb * CPS and nsb % 2 == 0 and CPS % 2 == 0

    @functools.partial(
        pl.kernel,
        out_type=jax.ShapeDtypeStruct((NC, n_pad, d), jnp.float32),
        mesh=_sc_mesh(),
        scratch_types=[
            pltpu.VMEM((2, CPS, EC), jnp.int32),
            pltpu.VMEM((2, CPS, EC), jnp.int32),
            pltpu.VMEM((2, EC, d), jnp.float32),
            pltpu.VMEM_SHARED((n_pad, d), jnp.float32),
            pltpu.SemaphoreType.DMA,
            pltpu.SemaphoreType.DMA,
            pltpu.SemaphoreType.DMA,
            pltpu.SemaphoreType.DMA,
            pltpu.SemaphoreType.DMA,
            pltpu.SemaphoreType.DMA,
        ],
    )
    def agg_k(h_hbm, ei_hbm, out_hbm, sibuf, dibuf, rows_v, acc_sh,
              isem0, isem1, gsem0, gsem1, ssem0, ssem1):
        isem = (isem0, isem1)
        gsem = (gsem0, gsem1)
        ssem = (ssem0, ssem1)
        cid = lax.axis_index("c")
        sid = lax.axis_index("s")
        wid = cid * NS + sid

        def zrow(r, _):
            for j in range(d // LANES):
                rows_v[0, r, pl.ds(j * LANES, LANES)] = jnp.zeros((LANES,), jnp.float32)
            return ()

        lax.fori_loop(0, EC, zrow, ())
        r0 = sid * rpt
        zc = EC
        off = 0
        while off < rpt:
            sz = min(zc, rpt - off)
            pltpu.sync_copy(rows_v.at[0, pl.ds(0, sz)],
                            acc_sh.at[pl.ds(r0 + off, sz)])
            off += sz
        plsc.subcore_barrier()

        def load_sb(sb, t):
            pltpu.async_copy(ei_hbm.at[0, wid, pl.ds(sb * CPS, CPS)], sibuf.at[t], isem[t])
            pltpu.async_copy(ei_hbm.at[1, wid, pl.ds(sb * CPS, CPS)], dibuf.at[t], isem[t])

        def wait_sb(t):
            pltpu.make_async_copy(ei_hbm.at[0, 0, pl.ds(0, CPS)], sibuf.at[t], isem[t]).wait()
            pltpu.make_async_copy(ei_hbm.at[0, 0, pl.ds(0, CPS)], dibuf.at[t], isem[t]).wait()

        def issue_gather(t, j, s):
            pltpu.async_copy(h_hbm.at[sibuf.at[t, j]], rows_v.at[s], gsem[s])

        def wait_gather(t, j, s):
            pltpu.make_async_copy(h_hbm.at[sibuf.at[t, j]], rows_v.at[s], gsem[s]).wait()

        def issue_scatter(t, j, s):
            pltpu.async_copy(rows_v.at[s], acc_sh.at[dibuf.at[t, j]], ssem[s], add=True)

        def wait_scatter(t, j, s):
            pltpu.make_async_copy(rows_v.at[s], acc_sh.at[dibuf.at[t, j]], ssem[s]).wait()

        def run_sb(sb, t, first_sb, last_sb):
            for j in range(CPS):
                s = j % 2
                first_k = first_sb and j == 0
                last_k = last_sb and j == CPS - 1
                if not first_k:
                    if j == 0:
                        wait_scatter(1 - t, CPS - 1, 1 - s)
                    else:
                        wait_scatter(t, j - 1, 1 - s)
                if not last_k:
                    if j == CPS - 1:
                        wait_sb(1 - t)
                        issue_gather(1 - t, 0, 1 - s)
                    else:
                        issue_gather(t, j + 1, 1 - s)
                wait_gather(t, j, s)
                issue_scatter(t, j, s)
                if j == 0 and not last_sb:
                    load_sb(sb + 1, 1 - t)

        pltpu.sync_copy(ei_hbm.at[0, wid, pl.ds(0, CPS)], sibuf.at[0])
        pltpu.sync_copy(ei_hbm.at[1, wid, pl.ds(0, CPS)], dibuf.at[0])
        issue_gather(0, 0, 0)
        run_sb(0, 0, True, False)

        def pair(q, _):
            run_sb(2 * q + 1, 1, False, False)
            run_sb(2 * q + 2, 0, False, False)
            return ()

        lax.fori_loop(0, (nsb - 2) // 2, pair, ())
        run_sb(nsb - 1, 1, False, True)
        wait_scatter(1, CPS - 1, (CPS - 1) % 2)
        plsc.subcore_barrier()
        pltpu.sync_copy(acc_sh.at[pl.ds(r0, rpt)], out_hbm.at[cid, pl.ds(r0, rpt)])

    return agg_k(h, ei4)


def _norms(degp, br=512):
    n_pad = degp.shape[2]
    eye = jnp.eye(br, dtype=jnp.float32)

    def body(deg_ref, eye_ref, ns_ref, nd_ref):
        dg = deg_ref[...]
        deg = jnp.clip(dg[0] + dg[1], 1.0, None)
        norm = lax.rsqrt(deg)
        cols = lax.dot_general(eye_ref[...], norm,
                               (((1,), (1,)), ((), ())),
                               precision=lax.Precision.HIGHEST,
                               preferred_element_type=jnp.float32)
        ns_ref[...] = cols[:, 0:1]
        nd_ref[...] = cols[:, 1:2]

    return pl.pallas_call(
        body,
        grid=(n_pad // br,),
        in_specs=[
            pl.BlockSpec((2, 2, br), lambda i: (0, 0, i)),
            pl.BlockSpec((br, br), lambda i: (0, 0)),
        ],
        out_specs=[
            pl.BlockSpec((br, 1), lambda i: (i, 0)),
            pl.BlockSpec((br, 1), lambda i: (i, 0)),
        ],
        out_shape=[
            jax.ShapeDtypeStruct((n_pad, 1), jnp.float32),
            jax.ShapeDtypeStruct((n_pad, 1), jnp.float32),
        ],
    )(degp, eye)


def _scale(x, ns_col, br=2000):
    n, d = x.shape

    def body(x_ref, ns_ref, h0_ref):
        h0_ref[...] = x_ref[...] * ns_ref[...]

    return pl.pallas_call(
        body,
        grid=(n // br,),
        in_specs=[
            pl.BlockSpec((br, d), lambda i: (i, 0)),
            pl.BlockSpec((br, 1), lambda i: (i, 0)),
        ],
        out_specs=pl.BlockSpec((br, d), lambda i: (i, 0)),
        out_shape=jax.ShapeDtypeStruct((n, d), jnp.float32),
    )(x, ns_col)


def _layer(aggp, nd_col, w, b2d, ns_col, relu, out_dtype, n, br=1000):
    d = w.shape[0]
    scaled = ns_col is not None

    def body(agg_ref, nd_ref, w_ref, b_ref, *rest):
        if scaled:
            ns_ref, o_ref = rest
        else:
            (o_ref,) = rest
        agg = (agg_ref[0] + agg_ref[1]) * nd_ref[...]
        h = jnp.dot(agg, w_ref[...], preferred_element_type=jnp.float32) + b_ref[...]
        if relu:
            h = jnp.maximum(h, 0.0)
        if scaled:
            h = h * ns_ref[...]
        o_ref[...] = h.astype(o_ref.dtype)

    in_specs = [
        pl.BlockSpec((NC, br, d), lambda i: (0, i, 0)),
        pl.BlockSpec((br, 1), lambda i: (i, 0)),
        pl.BlockSpec((d, d), lambda i: (0, 0)),
        pl.BlockSpec((1, d), lambda i: (0, 0)),
    ]
    args = [aggp, nd_col, w, b2d]
    if scaled:
        in_specs.append(pl.BlockSpec((br, 1), lambda i: (i, 0)))
        args.append(ns_col)
    return pl.pallas_call(
        body,
        grid=(n // br,),
        in_specs=in_specs,
        out_specs=pl.BlockSpec((br, d), lambda i: (i, 0)),
        out_shape=jax.ShapeDtypeStruct((n, d), out_dtype),
    )(*args)


def kernel(inputs, edge_index, W1, b1, W2, b2):
    x = inputs
    n, d = x.shape
    e = edge_index.shape[1]
    ei4 = edge_index.reshape(2, NW, e // (NW * EC), EC)
    n_pad = -(-n // (NS * LANES)) * (NS * LANES)

    degp = _degrees(ei4, n_pad)
    ns, nd = _norms(degp)
    h0 = _scale(x, ns)
    p1 = _aggregate(h0, ei4, n_pad)
    h1 = _layer(p1, nd, W1, b1.reshape(1, d), ns, relu=True,
                out_dtype=jnp.float32, n=n)
    p2 = _aggregate(h1, ei4, n_pad)
    out = _layer(p2, nd, W2, b2.reshape(1, d), None, relu=False,
                 out_dtype=jnp.float32, n=n)
    return out


# speedup vs baseline: 1.1720x; 1.0217x over previous
import functools

import jax
import jax.numpy as jnp
from jax import lax
from jax.experimental import pallas as pl
from jax.experimental.pallas import tpu as pltpu
from jax.experimental.pallas import tpu_sc as plsc

NC = 2
NS = 16
NW = NC * NS
LANES = 16
EC = 125
CPS = 8


def _sc_mesh():
    return plsc.VectorSubcoreMesh(core_axis_name="c", subcore_axis_name="s")


def _degrees(ei4, n_pad):
    _, _, nchunk, _ = ei4.shape
    rpt = n_pad // NS
    fs = 8
    assert nchunk % fs == 0

    @functools.partial(
        pl.kernel,
        out_type=jax.ShapeDtypeStruct((NC, 2, n_pad), jnp.float32),
        mesh=_sc_mesh(),
        scratch_types=[
            pltpu.VMEM((nchunk, EC), jnp.int32),
            pltpu.VMEM((nchunk, EC), jnp.int32),
            pltpu.VMEM((EC,), jnp.float32),
            pltpu.VMEM((rpt,), jnp.float32),
            pltpu.VMEM_SHARED((n_pad,), jnp.float32),
            pltpu.VMEM_SHARED((n_pad,), jnp.float32),
            pltpu.SemaphoreType.DMA,
        ],
    )
    def deg_k(ei_hbm, out_hbm, sidx_v, didx_v, ones_v, stage_v,
              dout_sh, din_sh, sem):
        cid = lax.axis_index("c")
        sid = lax.axis_index("s")
        wid = cid * NS + sid
        pltpu.sync_copy(ei_hbm.at[0, wid], sidx_v)
        pltpu.sync_copy(ei_hbm.at[1, wid], didx_v)
        for i in range(EC // LANES):
            ones_v[pl.ds(i * LANES, LANES)] = jnp.ones((LANES,), jnp.float32)
        ones_v[pl.ds(EC - LANES, LANES)] = jnp.ones((LANES,), jnp.float32)
        for i in range(rpt // LANES):
            stage_v[pl.ds(i * LANES, LANES)] = jnp.zeros((LANES,), jnp.float32)
        r0 = sid * rpt
        pltpu.sync_copy(stage_v, dout_sh.at[pl.ds(r0, rpt)])
        pltpu.sync_copy(stage_v, din_sh.at[pl.ds(r0, rpt)])
        plsc.subcore_barrier()

        def flight(f, _):
            cps = []
            for j in range(fs):
                k = f * fs + j
                cps.append(pltpu.async_copy(ones_v, dout_sh.at[sidx_v.at[k]], sem, add=True))
                cps.append(pltpu.async_copy(ones_v, din_sh.at[didx_v.at[k]], sem, add=True))
            for cp in cps:
                cp.wait()
            return ()

        lax.fori_loop(0, nchunk // fs, flight, ())
        plsc.subcore_barrier()
        pltpu.sync_copy(dout_sh.at[pl.ds(r0, rpt)], stage_v)
        pltpu.sync_copy(stage_v, out_hbm.at[cid, 0, pl.ds(r0, rpt)])
        pltpu.sync_copy(din_sh.at[pl.ds(r0, rpt)], stage_v)
        pltpu.sync_copy(stage_v, out_hbm.at[cid, 1, pl.ds(r0, rpt)])

    return deg_k(ei4)


def _aggregate(h, ei4, n_pad):
    n, d = h.shape
    _, _, nchunk, _ = ei4.shape
    nsb = nchunk // CPS
    rpt = n_pad // NS
    assert nchunk == nsb * CPS and nsb % 2 == 0 and CPS % 2 == 0

    @functools.partial(
        pl.kernel,
        out_type=jax.ShapeDtypeStruct((NC, n_pad, d), jnp.float32),
        mesh=_sc_mesh(),
        scratch_types=[
            pltpu.VMEM((2, CPS, EC), jnp.int32),
            pltpu.VMEM((2, CPS, EC), jnp.int32),
            pltpu.VMEM((2, EC, d), jnp.float32),
            pltpu.VMEM_SHARED((n_pad, d), jnp.float32),
            pltpu.SemaphoreType.DMA,
            pltpu.SemaphoreType.DMA,
            pltpu.SemaphoreType.DMA,
            pltpu.SemaphoreType.DMA,
            pltpu.SemaphoreType.DMA,
            pltpu.SemaphoreType.DMA,
        ],
    )
    def agg_k(h_hbm, ei_hbm, out_hbm, sibuf, dibuf, rows_v, acc_sh,
              isem0, isem1, gsem0, gsem1, ssem0, ssem1):
        isem = (isem0, isem1)
        gsem = (gsem0, gsem1)
        ssem = (ssem0, ssem1)
        cid = lax.axis_index("c")
        sid = lax.axis_index("s")
        wid = cid * NS + sid

        def zrow(r, _):
            for j in range(d // LANES):
                rows_v[0, r, pl.ds(j * LANES, LANES)] = jnp.zeros((LANES,), jnp.float32)
            return ()

        lax.fori_loop(0, EC, zrow, ())
        r0 = sid * rpt
        zc = EC
        off = 0
        while off < rpt:
            sz = min(zc, rpt - off)
            pltpu.sync_copy(rows_v.at[0, pl.ds(0, sz)],
                            acc_sh.at[pl.ds(r0 + off, sz)])
            off += sz
        plsc.subcore_barrier()

        def load_sb(sb, t):
            pltpu.async_copy(ei_hbm.at[0, wid, pl.ds(sb * CPS, CPS)], sibuf.at[t], isem[t])
            pltpu.async_copy(ei_hbm.at[1, wid, pl.ds(sb * CPS, CPS)], dibuf.at[t], isem[t])

        def wait_sb(t):
            pltpu.make_async_copy(ei_hbm.at[0, 0, pl.ds(0, CPS)], sibuf.at[t], isem[t]).wait()
            pltpu.make_async_copy(ei_hbm.at[0, 0, pl.ds(0, CPS)], dibuf.at[t], isem[t]).wait()

        def issue_gather(t, j, s):
            pltpu.async_copy(h_hbm.at[sibuf.at[t, j]], rows_v.at[s], gsem[s])

        def wait_gather(t, j, s):
            pltpu.make_async_copy(h_hbm.at[sibuf.at[t, j]], rows_v.at[s], gsem[s]).wait()

        def issue_scatter(t, j, s):
            pltpu.async_copy(rows_v.at[s], acc_sh.at[dibuf.at[t, j]], ssem[s], add=True)

        def wait_scatter(t, j, s):
            pltpu.make_async_copy(rows_v.at[s], acc_sh.at[dibuf.at[t, j]], ssem[s]).wait()

        def run_sb(sb, t, first_sb, last_sb):
            for j in range(CPS):
                s = j % 2
                first_k = first_sb and j == 0
                last_k = last_sb and j == CPS - 1
                if not first_k:
                    if j == 0:
                        wait_scatter(1 - t, CPS - 1, 1 - s)
                    else:
                        wait_scatter(t, j - 1, 1 - s)
                if not last_k:
                    if j == CPS - 1:
                        wait_sb(1 - t)
                        issue_gather(1 - t, 0, 1 - s)
                    else:
                        issue_gather(t, j + 1, 1 - s)
                wait_gather(t, j, s)
                issue_scatter(t, j, s)
                if j == 0 and not last_sb:
                    load_sb(sb + 1, 1 - t)

        pltpu.sync_copy(ei_hbm.at[0, wid, pl.ds(0, CPS)], sibuf.at[0])
        pltpu.sync_copy(ei_hbm.at[1, wid, pl.ds(0, CPS)], dibuf.at[0])
        issue_gather(0, 0, 0)
        run_sb(0, 0, True, False)

        def pair(q, _):
            run_sb(2 * q + 1, 1, False, False)
            run_sb(2 * q + 2, 0, False, False)
            return ()

        lax.fori_loop(0, (nsb - 2) // 2, pair, ())
        run_sb(nsb - 1, 1, False, True)
        wait_scatter(1, CPS - 1, (CPS - 1) % 2)
        plsc.subcore_barrier()
        pltpu.sync_copy(acc_sh.at[pl.ds(r0, rpt)], out_hbm.at[cid, pl.ds(r0, rpt)])

    return agg_k(h, ei4)


def _norms(degp, br=512):
    n_pad = degp.shape[2]
    eye = jnp.eye(br, dtype=jnp.float32)

    def body(deg_ref, eye_ref, ns_ref, nd_ref):
        dg = deg_ref[...]
        deg = jnp.clip(dg[0] + dg[1], 1.0, None)
        norm = lax.rsqrt(deg)
        cols = lax.dot_general(eye_ref[...], norm,
                               (((1,), (1,)), ((), ())),
                               precision=lax.Precision.HIGHEST,
                               preferred_element_type=jnp.float32)
        ns_ref[...] = cols[:, 0:1]
        nd_ref[...] = cols[:, 1:2]

    return pl.pallas_call(
        body,
        grid=(n_pad // br,),
        in_specs=[
            pl.BlockSpec((2, 2, br), lambda i: (0, 0, i)),
            pl.BlockSpec((br, br), lambda i: (0, 0)),
        ],
        out_specs=[
            pl.BlockSpec((br, 1), lambda i: (i, 0)),
            pl.BlockSpec((br, 1), lambda i: (i, 0)),
        ],
        out_shape=[
            jax.ShapeDtypeStruct((n_pad, 1), jnp.float32),
            jax.ShapeDtypeStruct((n_pad, 1), jnp.float32),
        ],
    )(degp, eye)


def _scale(x, ns_col, br=2000):
    n, d = x.shape

    def body(x_ref, ns_ref, h0_ref):
        h0_ref[...] = x_ref[...] * ns_ref[...]

    return pl.pallas_call(
        body,
        grid=(n // br,),
        in_specs=[
            pl.BlockSpec((br, d), lambda i: (i, 0)),
            pl.BlockSpec((br, 1), lambda i: (i, 0)),
        ],
        out_specs=pl.BlockSpec((br, d), lambda i: (i, 0)),
        out_shape=jax.ShapeDtypeStruct((n, d), jnp.float32),
    )(x, ns_col)


def _layer(aggp, nd_col, w, b2d, ns_col, relu, out_dtype, n, br=2000):
    d = w.shape[0]
    scaled = ns_col is not None

    def body(agg_ref, nd_ref, w_ref, b_ref, *rest):
        if scaled:
            ns_ref, o_ref = rest
        else:
            (o_ref,) = rest
        agg = (agg_ref[0] + agg_ref[1]) * nd_ref[...]
        h = jnp.dot(agg, w_ref[...], preferred_element_type=jnp.float32) + b_ref[...]
        if relu:
            h = jnp.maximum(h, 0.0)
        if scaled:
            h = h * ns_ref[...]
        o_ref[...] = h.astype(o_ref.dtype)

    in_specs = [
        pl.BlockSpec((NC, br, d), lambda i: (0, i, 0)),
        pl.BlockSpec((br, 1), lambda i: (i, 0)),
        pl.BlockSpec((d, d), lambda i: (0, 0)),
        pl.BlockSpec((1, d), lambda i: (0, 0)),
    ]
    args = [aggp, nd_col, w, b2d]
    if scaled:
        in_specs.append(pl.BlockSpec((br, 1), lambda i: (i, 0)))
        args.append(ns_col)
    return pl.pallas_call(
        body,
        grid=(n // br,),
        in_specs=in_specs,
        out_specs=pl.BlockSpec((br, d), lambda i: (i, 0)),
        out_shape=jax.ShapeDtypeStruct((n, d), out_dtype),
    )(*args)


def kernel(inputs, edge_index, W1, b1, W2, b2):
    x = inputs
    n, d = x.shape
    e = edge_index.shape[1]
    ei4 = edge_index.reshape(2, NW, e // (NW * EC), EC)
    n_pad = -(-n // (NS * LANES)) * (NS * LANES)

    degp = _degrees(ei4, n_pad)
    ns, nd = _norms(degp)
    h0 = _scale(x, ns)
    p1 = _aggregate(h0, ei4, n_pad)
    h1 = _layer(p1, nd, W1, b1.reshape(1, d), ns, relu=True,
                out_dtype=jnp.float32, n=n)
    p2 = _aggregate(h1, ei4, n_pad)
    out = _layer(p2, nd, W2, b2.reshape(1, d), None, relu=False,
                 out_dtype=jnp.float32, n=n)
    return out


# speedup vs baseline: 1.1902x; 1.0155x over previous
import functools

import jax
import jax.numpy as jnp
from jax import lax
from jax.experimental import pallas as pl
from jax.experimental.pallas import tpu as pltpu
from jax.experimental.pallas import tpu_sc as plsc

NC = 2
NS = 16
NW = NC * NS
LANES = 16
EC = 125
CPS = 8


def _sc_mesh():
    return plsc.VectorSubcoreMesh(core_axis_name="c", subcore_axis_name="s")


def _degrees(ei4, n_pad):
    _, _, nchunk, _ = ei4.shape
    rpt = n_pad // NS
    fs = 8
    assert nchunk % fs == 0

    @functools.partial(
        pl.kernel,
        out_type=jax.ShapeDtypeStruct((NC, 2, n_pad), jnp.float32),
        mesh=_sc_mesh(),
        scratch_types=[
            pltpu.VMEM((nchunk, EC), jnp.int32),
            pltpu.VMEM((nchunk, EC), jnp.int32),
            pltpu.VMEM((EC,), jnp.float32),
            pltpu.VMEM((rpt,), jnp.float32),
            pltpu.VMEM_SHARED((n_pad,), jnp.float32),
            pltpu.VMEM_SHARED((n_pad,), jnp.float32),
            pltpu.SemaphoreType.DMA,
        ],
    )
    def deg_k(ei_hbm, out_hbm, sidx_v, didx_v, ones_v, stage_v,
              dout_sh, din_sh, sem):
        cid = lax.axis_index("c")
        sid = lax.axis_index("s")
        wid = cid * NS + sid
        pltpu.sync_copy(ei_hbm.at[0, wid], sidx_v)
        pltpu.sync_copy(ei_hbm.at[1, wid], didx_v)
        for i in range(EC // LANES):
            ones_v[pl.ds(i * LANES, LANES)] = jnp.ones((LANES,), jnp.float32)
        ones_v[pl.ds(EC - LANES, LANES)] = jnp.ones((LANES,), jnp.float32)
        for i in range(rpt // LANES):
            stage_v[pl.ds(i * LANES, LANES)] = jnp.zeros((LANES,), jnp.float32)
        r0 = sid * rpt
        pltpu.sync_copy(stage_v, dout_sh.at[pl.ds(r0, rpt)])
        pltpu.sync_copy(stage_v, din_sh.at[pl.ds(r0, rpt)])
        plsc.subcore_barrier()

        def flight(f, _):
            cps = []
            for j in range(fs):
                k = f * fs + j
                cps.append(pltpu.async_copy(ones_v, dout_sh.at[sidx_v.at[k]], sem, add=True))
                cps.append(pltpu.async_copy(ones_v, din_sh.at[didx_v.at[k]], sem, add=True))
            for cp in cps:
                cp.wait()
            return ()

        lax.fori_loop(0, nchunk // fs, flight, ())
        plsc.subcore_barrier()
        pltpu.sync_copy(dout_sh.at[pl.ds(r0, rpt)], stage_v)
        pltpu.sync_copy(stage_v, out_hbm.at[cid, 0, pl.ds(r0, rpt)])
        pltpu.sync_copy(din_sh.at[pl.ds(r0, rpt)], stage_v)
        pltpu.sync_copy(stage_v, out_hbm.at[cid, 1, pl.ds(r0, rpt)])

    return deg_k(ei4)


def _aggregate(h, ei4, n_pad):
    n, d = h.shape
    _, _, nchunk, _ = ei4.shape
    nsb = nchunk // CPS
    rpt = n_pad // NS
    assert nchunk == nsb * CPS and nsb % 2 == 0 and CPS % 2 == 0

    @functools.partial(
        pl.kernel,
        out_type=jax.ShapeDtypeStruct((NC, n_pad, d), jnp.float32),
        mesh=_sc_mesh(),
        scratch_types=[
            pltpu.VMEM((2, CPS, EC), jnp.int32),
            pltpu.VMEM((2, CPS, EC), jnp.int32),
            pltpu.VMEM((2, EC, d), jnp.float32),
            pltpu.VMEM_SHARED((n_pad, d), jnp.float32),
            pltpu.SemaphoreType.DMA,
            pltpu.SemaphoreType.DMA,
            pltpu.SemaphoreType.DMA,
            pltpu.SemaphoreType.DMA,
            pltpu.SemaphoreType.DMA,
            pltpu.SemaphoreType.DMA,
        ],
    )
    def agg_k(h_hbm, ei_hbm, out_hbm, sibuf, dibuf, rows_v, acc_sh,
              isem0, isem1, gsem0, gsem1, ssem0, ssem1):
        isem = (isem0, isem1)
        gsem = (gsem0, gsem1)
        ssem = (ssem0, ssem1)
        cid = lax.axis_index("c")
        sid = lax.axis_index("s")
        wid = cid * NS + sid

        def zrow(r, _):
            for j in range(d // LANES):
                rows_v[0, r, pl.ds(j * LANES, LANES)] = jnp.zeros((LANES,), jnp.float32)
            return ()

        lax.fori_loop(0, EC, zrow, ())
        r0 = sid * rpt
        zc = EC
        off = 0
        while off < rpt:
            sz = min(zc, rpt - off)
            pltpu.sync_copy(rows_v.at[0, pl.ds(0, sz)],
                            acc_sh.at[pl.ds(r0 + off, sz)])
            off += sz
        plsc.subcore_barrier()

        def load_sb(sb, t):
            pltpu.async_copy(ei_hbm.at[0, wid, pl.ds(sb * CPS, CPS)], sibuf.at[t], isem[t])
            pltpu.async_copy(ei_hbm.at[1, wid, pl.ds(sb * CPS, CPS)], dibuf.at[t], isem[t])

        def wait_sb(t):
            pltpu.make_async_copy(ei_hbm.at[0, 0, pl.ds(0, CPS)], sibuf.at[t], isem[t]).wait()
            pltpu.make_async_copy(ei_hbm.at[0, 0, pl.ds(0, CPS)], dibuf.at[t], isem[t]).wait()

        def issue_gather(t, j, s):
            pltpu.async_copy(h_hbm.at[sibuf.at[t, j]], rows_v.at[s], gsem[s])

        def wait_gather(t, j, s):
            pltpu.make_async_copy(h_hbm.at[sibuf.at[t, j]], rows_v.at[s], gsem[s]).wait()

        def issue_scatter(t, j, s):
            pltpu.async_copy(rows_v.at[s], acc_sh.at[dibuf.at[t, j]], ssem[s], add=True)

        def wait_scatter(t, j, s):
            pltpu.make_async_copy(rows_v.at[s], acc_sh.at[dibuf.at[t, j]], ssem[s]).wait()

        def run_sb(sb, t, first_sb, last_sb):
            for j in range(CPS):
                s = j % 2
                first_k = first_sb and j == 0
                last_k = last_sb and j == CPS - 1
                if not first_k:
                    if j == 0:
                        wait_scatter(1 - t, CPS - 1, 1 - s)
                    else:
                        wait_scatter(t, j - 1, 1 - s)
                if not last_k:
                    if j == CPS - 1:
                        wait_sb(1 - t)
                        issue_gather(1 - t, 0, 1 - s)
                    else:
                        issue_gather(t, j + 1, 1 - s)
                wait_gather(t, j, s)
                issue_scatter(t, j, s)
                if j == 0 and not last_sb:
                    load_sb(sb + 1, 1 - t)

        pltpu.sync_copy(ei_hbm.at[0, wid, pl.ds(0, CPS)], sibuf.at[0])
        pltpu.sync_copy(ei_hbm.at[1, wid, pl.ds(0, CPS)], dibuf.at[0])
        issue_gather(0, 0, 0)
        run_sb(0, 0, True, False)

        def pair(q, _):
            run_sb(2 * q + 1, 1, False, False)
            run_sb(2 * q + 2, 0, False, False)
            return ()

        lax.fori_loop(0, (nsb - 2) // 2, pair, ())
        run_sb(nsb - 1, 1, False, True)
        wait_scatter(1, CPS - 1, (CPS - 1) % 2)
        plsc.subcore_barrier()
        pltpu.sync_copy(acc_sh.at[pl.ds(r0, rpt)], out_hbm.at[cid, pl.ds(r0, rpt)])

    return agg_k(h, ei4)


def _norms(degp, br=512):
    n_pad = degp.shape[2]
    eye = jnp.eye(br, dtype=jnp.float32)

    def body(deg_ref, eye_ref, ns_ref, nd_ref):
        dg = deg_ref[...]
        deg = jnp.clip(dg[0] + dg[1], 1.0, None)
        norm = lax.rsqrt(deg)
        cols = lax.dot_general(eye_ref[...], norm,
                               (((1,), (1,)), ((), ())),
                               precision=lax.Precision.HIGHEST,
                               preferred_element_type=jnp.float32)
        ns_ref[...] = cols[:, 0:1]
        nd_ref[...] = cols[:, 1:2]

    return pl.pallas_call(
        body,
        grid=(n_pad // br,),
        in_specs=[
            pl.BlockSpec((2, 2, br), lambda i: (0, 0, i)),
            pl.BlockSpec((br, br), lambda i: (0, 0)),
        ],
        out_specs=[
            pl.BlockSpec((br, 1), lambda i: (i, 0)),
            pl.BlockSpec((br, 1), lambda i: (i, 0)),
        ],
        out_shape=[
            jax.ShapeDtypeStruct((n_pad, 1), jnp.float32),
            jax.ShapeDtypeStruct((n_pad, 1), jnp.float32),
        ],
    )(degp, eye)


def _scale(x, ns_col, br=5000):
    n, d = x.shape

    def body(x_ref, ns_ref, h0_ref):
        h0_ref[...] = x_ref[...] * ns_ref[...]

    return pl.pallas_call(
        body,
        grid=(n // br,),
        in_specs=[
            pl.BlockSpec((br, d), lambda i: (i, 0)),
            pl.BlockSpec((br, 1), lambda i: (i, 0)),
        ],
        out_specs=pl.BlockSpec((br, d), lambda i: (i, 0)),
        out_shape=jax.ShapeDtypeStruct((n, d), jnp.float32),
    )(x, ns_col)


def _layer(aggp, nd_col, w, b2d, ns_col, relu, out_dtype, n, br=5000):
    d = w.shape[0]
    scaled = ns_col is not None

    def body(agg_ref, nd_ref, w_ref, b_ref, *rest):
        if scaled:
            ns_ref, o_ref = rest
        else:
            (o_ref,) = rest
        agg = (agg_ref[0] + agg_ref[1]) * nd_ref[...]
        h = jnp.dot(agg, w_ref[...], preferred_element_type=jnp.float32) + b_ref[...]
        if relu:
            h = jnp.maximum(h, 0.0)
        if scaled:
            h = h * ns_ref[...]
        o_ref[...] = h.astype(o_ref.dtype)

    in_specs = [
        pl.BlockSpec((NC, br, d), lambda i: (0, i, 0)),
        pl.BlockSpec((br, 1), lambda i: (i, 0)),
        pl.BlockSpec((d, d), lambda i: (0, 0)),
        pl.BlockSpec((1, d), lambda i: (0, 0)),
    ]
    args = [aggp, nd_col, w, b2d]
    if scaled:
        in_specs.append(pl.BlockSpec((br, 1), lambda i: (i, 0)))
        args.append(ns_col)
    return pl.pallas_call(
        body,
        grid=(n // br,),
        in_specs=in_specs,
        out_specs=pl.BlockSpec((br, d), lambda i: (i, 0)),
        out_shape=jax.ShapeDtypeStruct((n, d), out_dtype),
    )(*args)


def kernel(inputs, edge_index, W1, b1, W2, b2):
    x = inputs
    n, d = x.shape
    e = edge_index.shape[1]
    ei4 = edge_index.reshape(2, NW, e // (NW * EC), EC)
    n_pad = -(-n // (NS * LANES)) * (NS * LANES)

    degp = _degrees(ei4, n_pad)
    ns, nd = _norms(degp)
    h0 = _scale(x, ns)
    p1 = _aggregate(h0, ei4, n_pad)
    h1 = _layer(p1, nd, W1, b1.reshape(1, d), ns, relu=True,
                out_dtype=jnp.float32, n=n)
    p2 = _aggregate(h1, ei4, n_pad)
    out = _layer(p2, nd, W2, b2.reshape(1, d), None, relu=False,
                 out_dtype=jnp.float32, n=n)
    return out
